# Initial kernel scaffold; baseline (speedup 1.0000x reference)
import jax
import jax.numpy as jnp
from jax.experimental import pallas as pl


def kernel(x, edge_index, edge_attr, params):
    raise NotImplementedError("write your pallas kernel here")


# speedup vs baseline: 1.5075x; 1.5075x over previous
import functools

import jax
import jax.numpy as jnp
from jax import lax
from jax.experimental import pallas as pl
from jax.experimental.pallas import tpu as pltpu
from jax.experimental.pallas import tpu_sc as plsc

_N = 10000
_E = 320000
_L = 32
_NC, _NS = 2, 16
_NW = _NC * _NS

_E_PAD = 327680
_IDX_PAD = 655360
_EB = 512
_N_EBLK = _E // _EB
_N_EBLK_PAD = _E_PAD // _EB


def _ln(y, g, be):
    mu = jnp.mean(y, axis=-1, keepdims=True)
    var = jnp.mean((y - mu) ** 2, axis=-1, keepdims=True)
    return (y - mu) * lax.rsqrt(var + 1e-5) * g + be


def _mlp(y, w0, b0, w1, b1, w2, b2):
    y = jnp.maximum(jnp.dot(y, w0, preferred_element_type=jnp.float32) + b0, 0.0)
    y = jnp.maximum(jnp.dot(y, w1, preferred_element_type=jnp.float32) + b1, 0.0)
    return jnp.dot(y, w2, preferred_element_type=jnp.float32) + b2


def _flat(p, with_ln=True):
    (w0, b0), (w1, b1), (w2, b2) = p["layers"]
    out = [w0, b0.reshape(1, -1), w1, b1.reshape(1, -1), w2, b2.reshape(1, -1)]
    if with_ln:
        g, be = p["ln"]
        out += [g.reshape(1, -1), be.reshape(1, -1)]
    return out


def _enc_node(x, ws):
    def body(x_ref, w0, b0, w1, b1, w2, b2, g, be, o_ref):
        y = _mlp(x_ref[...], w0[...], b0[...], w1[...], b1[...], w2[...], b2[...])
        o_ref[...] = _ln(y, g[...], be[...])

    return pl.pallas_call(
        body, out_shape=jax.ShapeDtypeStruct((_N, _L), jnp.float32))(x, *ws)


def _enc_edge(ea, ws):
    def body(x_ref, w0, b0, w1, b1, w2, b2, g, be, o_ref):
        pid = pl.program_id(0)
        y = _mlp(x_ref[...], w0[...], b0[...], w1[...], b1[...], w2[...], b2[...])
        y = _ln(y, g[...], be[...])
        o_ref[...] = jnp.where(pid < _N_EBLK, y, 0.0)

    wspecs = [pl.BlockSpec(w.shape, lambda i: (0, 0)) for w in ws]
    return pl.pallas_call(
        body,
        grid=(_N_EBLK_PAD,),
        in_specs=[pl.BlockSpec((_EB, 4), lambda i: (jnp.minimum(i, _N_EBLK - 1), 0))]
        + wspecs,
        out_specs=pl.BlockSpec((_EB, _L), lambda i: (i, 0)),
        out_shape=jax.ShapeDtypeStruct((_E_PAD, _L), jnp.float32),
    )(ea, *ws)


def _edge_step(e, gath, ws):
    def body(e_ref, hs_ref, hd_ref, w0, b0, w1, b1, w2, b2, g, be, o_ref):
        pid = pl.program_id(0)
        xcat = jnp.concatenate([e_ref[...], hs_ref[...], hd_ref[...]], axis=-1)
        y = _mlp(xcat, w0[...], b0[...], w1[...], b1[...], w2[...], b2[...])
        y = _ln(y, g[...], be[...])
        o_ref[...] = jnp.where(pid < _N_EBLK, e_ref[...] + y, 0.0)

    wspecs = [pl.BlockSpec(w.shape, lambda i: (0, 0)) for w in ws]
    return pl.pallas_call(
        body,
        grid=(_N_EBLK_PAD,),
        in_specs=[
            pl.BlockSpec((_EB, _L), lambda i: (i, 0)),
            pl.BlockSpec((_EB, _L), lambda i: (i, 0)),
            pl.BlockSpec((_EB, _L), lambda i: (i + _N_EBLK, 0)),
        ]
        + wspecs,
        out_specs=pl.BlockSpec((_EB, _L), lambda i: (i, 0)),
        out_shape=jax.ShapeDtypeStruct((_E_PAD, _L), jnp.float32),
    )(e, gath, gath, *ws)


def _node_step(h, agg2, ws):
    def body(h_ref, a_ref, w0, b0, w1, b1, w2, b2, g, be, o_ref):
        a = a_ref[0] + a_ref[1]
        y = _mlp(jnp.concatenate([h_ref[...], a], axis=-1),
                 w0[...], b0[...], w1[...], b1[...], w2[...], b2[...])
        o_ref[...] = h_ref[...] + _ln(y, g[...], be[...])

    return pl.pallas_call(
        body, out_shape=jax.ShapeDtypeStruct((_N, _L), jnp.float32))(h, agg2, *ws)


def _dec(h, ws):
    def body(h_ref, w0, b0, w1, b1, w2, b2, o_ref):
        o_ref[...] = _mlp(h_ref[...], w0[...], b0[...], w1[...], b1[...], w2[...], b2[...])

    return pl.pallas_call(
        body, out_shape=jax.ShapeDtypeStruct((_N, 3), jnp.float32))(h, *ws)


def _sc_mesh():
    return plsc.VectorSubcoreMesh(core_axis_name="c", subcore_axis_name="s")


def _sc_gather(h, idx2d):
    @functools.partial(
        pl.kernel,
        mesh=_sc_mesh(),
        compiler_params=pltpu.CompilerParams(use_tc_tiling_on_sc=False),
        out_type=jax.ShapeDtypeStruct((_IDX_PAD, _L), jnp.float32),
        scratch_types=[
            pltpu.VMEM((8, 128), jnp.int32),
            pltpu.VMEM((1024, _L), jnp.float32),
            pltpu.SemaphoreType.DMA,
        ],
    )
    def gk(h_hbm, idx_hbm, out_hbm, idx_v, rows_v, sem):
        wid = lax.axis_index("s") * _NC + lax.axis_index("c")

        def step(t, c):
            pltpu.sync_copy(idx_hbm.at[pl.ds(wid * 160 + t * 8, 8)], idx_v)
            cps = [
                pltpu.async_copy(
                    h_hbm.at[idx_v.at[j]], rows_v.at[pl.ds(j * 128, 128)], sem)
                for j in range(8)
            ]
            for cp in cps:
                cp.wait()
            pltpu.sync_copy(rows_v, out_hbm.at[pl.ds(wid * 20480 + t * 1024, 1024)])
            return c

        lax.fori_loop(0, 20, step, 0)

    return gk(h, idx2d)


def _sc_scatter(e, didx2d, zblk):
    @functools.partial(
        pl.kernel,
        mesh=_sc_mesh(),
        compiler_params=pltpu.CompilerParams(use_tc_tiling_on_sc=False),
        out_type=jax.ShapeDtypeStruct((2 * _N, _L), jnp.float32),
        scratch_types=[
            pltpu.VMEM_SHARED((10240, _L), jnp.float32),
            pltpu.VMEM((8, 128), jnp.int32),
            pltpu.VMEM((1024, _L), jnp.float32),
            pltpu.SemaphoreType.DMA,
        ],
    )
    def sk(e_hbm, idx_hbm, z_hbm, out_hbm, shared, idx_v, rows_v, sem):
        core = lax.axis_index("c")
        sub = lax.axis_index("s")
        wid = sub * _NC + core
        pltpu.sync_copy(z_hbm, shared.at[pl.ds(sub * 640, 640)])
        plsc.subcore_barrier()

        def step(t, c):
            pltpu.sync_copy(e_hbm.at[pl.ds(wid * 10240 + t * 1024, 1024)], rows_v)
            pltpu.sync_copy(idx_hbm.at[pl.ds(wid * 80 + t * 8, 8)], idx_v)
            for j in range(8):
                pltpu.sync_copy(
                    rows_v.at[pl.ds(j * 128, 128)],
                    shared.at[idx_v.at[j]],
                    add=True,
                )
            return c

        lax.fori_loop(0, 10, step, 0)
        plsc.subcore_barrier()
        pltpu.sync_copy(
            shared.at[pl.ds(sub * 625, 625)],
            out_hbm.at[pl.ds(core * _N + sub * 625, 625)],
        )

    return sk(e, didx2d, zblk).reshape(2, _N, _L)


def kernel(x, edge_index, edge_attr, params):
    gidx = jnp.concatenate(
        [edge_index.reshape(-1), jnp.zeros((_IDX_PAD - 2 * _E,), jnp.int32)]
    ).reshape(_IDX_PAD // 128, 128)
    didx = jnp.concatenate(
        [edge_index[1], jnp.zeros((_E_PAD - _E,), jnp.int32)]
    ).reshape(_E_PAD // 128, 128)
    zblk = jnp.zeros((640, _L), jnp.float32)

    h = _enc_node(x, _flat(params["enc_node"]))
    e = _enc_edge(edge_attr, _flat(params["enc_edge"]))
    for blk in params["processor"]:
        gath = _sc_gather(h, gidx)
        e = _edge_step(e, gath, _flat(blk["edge"]))
        agg2 = _sc_scatter(e, didx, zblk)
        h = _node_step(h, agg2, _flat(blk["node"]))
    return _dec(h, _flat(params["dec"], with_ln=False))


# speedup vs baseline: 4.4957x; 2.9822x over previous
import functools

import jax
import jax.numpy as jnp
from jax import lax
from jax.experimental import pallas as pl
from jax.experimental.pallas import tpu as pltpu
from jax.experimental.pallas import tpu_sc as plsc

_N = 10000
_E = 320000
_L = 32
_NC, _NS = 2, 16
_NW = _NC * _NS

_E_PAD = 327680
_EP = _E_PAD // 4
_NP = _N // 4
_IDX_PAD = 2 * _E_PAD
_EBP = 1024
_N_EBLK = _EP // _EBP
_VALID_P = _E // 4


def _ln_packed(x, g, be, m):
    mu = jnp.dot(x, m, preferred_element_type=jnp.float32)
    xc = x - mu
    var = jnp.dot(xc * xc, m, preferred_element_type=jnp.float32)
    return xc * lax.rsqrt(var + 1e-5) * g + be


def _kron4(w):
    return jnp.kron(jnp.eye(4, dtype=jnp.float32), w)


def _tile4(v):
    return jnp.tile(v, 4).reshape(1, -1)


def _mavg():
    return _kron4(jnp.full((_L, _L), 1.0 / _L, jnp.float32))


def _prep(p, splits, with_ln=True):
    layers = p["layers"]
    (w0, b0) = layers[0]
    out = [_kron4(w) for w in jnp.split(w0, splits, axis=0)] + [_tile4(b0)]
    for w, b in layers[1:]:
        out += [_kron4(w), _tile4(b)]
    if with_ln:
        g, be = p["ln"]
        out += [_tile4(g), _tile4(be), _mavg()]
    return out


def _enc_node(x4, ws):
    def body(x_ref, d0, b0, d1, b1, d2, b2, g, be, m, o_ref):
        y = jnp.maximum(
            jnp.dot(x_ref[...], d0[...], preferred_element_type=jnp.float32)
            + b0[...], 0.0)
        y = jnp.maximum(
            jnp.dot(y, d1[...], preferred_element_type=jnp.float32) + b1[...], 0.0)
        y = jnp.dot(y, d2[...], preferred_element_type=jnp.float32) + b2[...]
        o_ref[...] = _ln_packed(y, g[...], be[...], m[...])

    return pl.pallas_call(
        body, out_shape=jax.ShapeDtypeStruct((_NP, 128), jnp.float32))(x4, *ws)


def _enc_edge(ea16, ws):
    def body(x_ref, d0, b0, d1, b1, d2, b2, g, be, m, o_ref):
        i = pl.program_id(0)
        y = jnp.maximum(
            jnp.dot(x_ref[...], d0[...], preferred_element_type=jnp.float32)
            + b0[...], 0.0)
        y = jnp.maximum(
            jnp.dot(y, d1[...], preferred_element_type=jnp.float32) + b1[...], 0.0)
        y = jnp.dot(y, d2[...], preferred_element_type=jnp.float32) + b2[...]
        y = _ln_packed(y, g[...], be[...], m[...])
        rows = i * _EBP + lax.broadcasted_iota(jnp.int32, (_EBP, 128), 0)
        o_ref[...] = jnp.where(rows < _VALID_P, y, 0.0)

    wspecs = [pl.BlockSpec(w.shape, lambda i: (0, 0)) for w in ws]
    return pl.pallas_call(
        body,
        grid=(_N_EBLK,),
        in_specs=[pl.BlockSpec((_EBP, 16), lambda i: (i, 0))] + wspecs,
        out_specs=pl.BlockSpec((_EBP, 128), lambda i: (i, 0)),
        out_shape=jax.ShapeDtypeStruct((_EP, 128), jnp.float32),
    )(ea16, *ws)


def _edge_step(e_p, gath_p, ws):
    def body(e_ref, hs_ref, hd_ref, d1e, d1s, d1d, b1, d2, b2, d3, b3,
             g, be, m, o_ref):
        i = pl.program_id(0)
        y = jnp.dot(e_ref[...], d1e[...], preferred_element_type=jnp.float32)
        y = y + jnp.dot(hs_ref[...], d1s[...], preferred_element_type=jnp.float32)
        y = y + jnp.dot(hd_ref[...], d1d[...], preferred_element_type=jnp.float32)
        y = jnp.maximum(y + b1[...], 0.0)
        y = jnp.maximum(
            jnp.dot(y, d2[...], preferred_element_type=jnp.float32) + b2[...], 0.0)
        y = jnp.dot(y, d3[...], preferred_element_type=jnp.float32) + b3[...]
        y = _ln_packed(y, g[...], be[...], m[...])
        rows = i * _EBP + lax.broadcasted_iota(jnp.int32, (_EBP, 128), 0)
        o_ref[...] = jnp.where(rows < _VALID_P, e_ref[...] + y, 0.0)

    wspecs = [pl.BlockSpec(w.shape, lambda i: (0, 0)) for w in ws]
    return pl.pallas_call(
        body,
        grid=(_N_EBLK,),
        in_specs=[
            pl.BlockSpec((_EBP, 128), lambda i: (i, 0)),
            pl.BlockSpec((_EBP, 128), lambda i: (i, 0)),
            pl.BlockSpec((_EBP, 128), lambda i: (i + _N_EBLK, 0)),
        ]
        + wspecs,
        out_specs=pl.BlockSpec((_EBP, 128), lambda i: (i, 0)),
        out_shape=jax.ShapeDtypeStruct((_EP, 128), jnp.float32),
    )(e_p, gath_p, gath_p, *ws)


def _node_step(h_p, agg_p, ws):
    def body(h_ref, a_ref, d1h, d1a, b1, d2, b2, d3, b3, g, be, m, o_ref):
        a = a_ref[pl.ds(0, _NP), :] + a_ref[pl.ds(_NP, _NP), :]
        y = jnp.dot(h_ref[...], d1h[...], preferred_element_type=jnp.float32)
        y = y + jnp.dot(a, d1a[...], preferred_element_type=jnp.float32)
        y = jnp.maximum(y + b1[...], 0.0)
        y = jnp.maximum(
            jnp.dot(y, d2[...], preferred_element_type=jnp.float32) + b2[...], 0.0)
        y = jnp.dot(y, d3[...], preferred_element_type=jnp.float32) + b3[...]
        o_ref[...] = h_ref[...] + _ln_packed(y, g[...], be[...], m[...])

    return pl.pallas_call(
        body, out_shape=jax.ShapeDtypeStruct((_NP, 128), jnp.float32))(
            h_p, agg_p, *ws)


def _dec(h_p, ws):
    def body(h_ref, d0, b0, d1, b1, d2, b2, o_ref):
        y = jnp.maximum(
            jnp.dot(h_ref[...], d0[...], preferred_element_type=jnp.float32)
            + b0[...], 0.0)
        y = jnp.maximum(
            jnp.dot(y, d1[...], preferred_element_type=jnp.float32) + b1[...], 0.0)
        o_ref[...] = jnp.dot(y, d2[...], preferred_element_type=jnp.float32) + b2[...]

    return pl.pallas_call(
        body, out_shape=jax.ShapeDtypeStruct((_NP, 12), jnp.float32))(h_p, *ws)


def _sc_mesh():
    return plsc.VectorSubcoreMesh(core_axis_name="c", subcore_axis_name="s")


def _sc_gather(h, idx2d):
    @functools.partial(
        pl.kernel,
        mesh=_sc_mesh(),
        compiler_params=pltpu.CompilerParams(use_tc_tiling_on_sc=False),
        out_type=jax.ShapeDtypeStruct((_IDX_PAD, _L), jnp.float32),
        scratch_types=[
            pltpu.VMEM((8, 128), jnp.int32),
            pltpu.VMEM((1024, _L), jnp.float32),
            pltpu.SemaphoreType.DMA,
        ],
    )
    def gk(h_hbm, idx_hbm, out_hbm, idx_v, rows_v, sem):
        wid = lax.axis_index("s") * _NC + lax.axis_index("c")

        def step(t, c):
            pltpu.sync_copy(idx_hbm.at[pl.ds(wid * 160 + t * 8, 8)], idx_v)
            cps = [
                pltpu.async_copy(
                    h_hbm.at[idx_v.at[j]], rows_v.at[pl.ds(j * 128, 128)], sem)
                for j in range(8)
            ]
            for cp in cps:
                cp.wait()
            pltpu.sync_copy(rows_v, out_hbm.at[pl.ds(wid * 20480 + t * 1024, 1024)])
            return c

        lax.fori_loop(0, 20, step, 0)

    return gk(h, idx2d)


def _sc_scatter(e, didx2d, zblk):
    @functools.partial(
        pl.kernel,
        mesh=_sc_mesh(),
        compiler_params=pltpu.CompilerParams(use_tc_tiling_on_sc=False),
        out_type=jax.ShapeDtypeStruct((2 * _N, _L), jnp.float32),
        scratch_types=[
            pltpu.VMEM_SHARED((10240, _L), jnp.float32),
            pltpu.VMEM((8, 128), jnp.int32),
            pltpu.VMEM((1024, _L), jnp.float32),
            pltpu.SemaphoreType.DMA,
        ],
    )
    def sk(e_hbm, idx_hbm, z_hbm, out_hbm, shared, idx_v, rows_v, sem):
        core = lax.axis_index("c")
        sub = lax.axis_index("s")
        wid = sub * _NC + core
        pltpu.sync_copy(z_hbm, shared.at[pl.ds(sub * 640, 640)])
        plsc.subcore_barrier()

        def step(t, c):
            pltpu.sync_copy(e_hbm.at[pl.ds(wid * 10240 + t * 1024, 1024)], rows_v)
            pltpu.sync_copy(idx_hbm.at[pl.ds(wid * 80 + t * 8, 8)], idx_v)
            for j in range(8):
                pltpu.sync_copy(
                    rows_v.at[pl.ds(j * 128, 128)],
                    shared.at[idx_v.at[j]],
                    add=True,
                )
            return c

        lax.fori_loop(0, 10, step, 0)
        plsc.subcore_barrier()
        pltpu.sync_copy(
            shared.at[pl.ds(sub * 625, 625)],
            out_hbm.at[pl.ds(core * _N + sub * 625, 625)],
        )

    return sk(e, didx2d, zblk)


def kernel(x, edge_index, edge_attr, params):
    zpad = jnp.zeros((_E_PAD - _E,), jnp.int32)
    gidx = jnp.concatenate(
        [edge_index[0], zpad, edge_index[1], zpad]).reshape(_IDX_PAD // 128, 128)
    didx = jnp.concatenate([edge_index[1], zpad]).reshape(_E_PAD // 128, 128)
    zblk = jnp.zeros((640, _L), jnp.float32)

    h_p = _enc_node(x.reshape(_NP, 512), _prep(params["enc_node"], 1))
    ea16 = jnp.pad(edge_attr, ((0, _E_PAD - _E), (0, 0))).reshape(_EP, 16)
    e_p = _enc_edge(ea16, _prep(params["enc_edge"], 1))
    for blk in params["processor"]:
        ews = _prep(blk["edge"], 3)
        nws = _prep(blk["node"], 2)
        gath = _sc_gather(h_p.reshape(_N, _L), gidx)
        e_p = _edge_step(e_p, gath.reshape(2 * _EP, 128), ews)
        agg = _sc_scatter(e_p.reshape(_E_PAD, _L), didx, zblk)
        h_p = _node_step(h_p, agg.reshape(2 * _NP, 128), nws)
    return _dec(h_p, _prep(params["dec"], 1, with_ln=False)).reshape(_N, 3)


# speedup vs baseline: 4.9623x; 1.1038x over previous
import functools

import jax
import jax.numpy as jnp
from jax import lax
from jax.experimental import pallas as pl
from jax.experimental.pallas import tpu as pltpu
from jax.experimental.pallas import tpu_sc as plsc

_N = 10000
_E = 320000
_L = 32
_NC, _NS = 2, 16
_NW = _NC * _NS

_E_PAD = 327680
_EP = _E_PAD // 4
_NP = _N // 4
_IDX_PAD = 2 * _E_PAD
_EBP = 2048
_N_EBLK = _EP // _EBP
_VALID_P = _E // 4


def _ln_packed(x, g, be, m):
    mu = jnp.dot(x, m, preferred_element_type=jnp.float32)
    xc = x - mu
    var = jnp.dot(xc * xc, m, preferred_element_type=jnp.float32)
    return xc * lax.rsqrt(var + 1e-5) * g + be


def _kron4(w):
    return jnp.kron(jnp.eye(4, dtype=jnp.float32), w)


def _tile4(v):
    return jnp.tile(v, 4).reshape(1, -1)


def _mavg():
    return _kron4(jnp.full((_L, _L), 1.0 / _L, jnp.float32))


def _prep(p, splits, with_ln=True):
    layers = p["layers"]
    (w0, b0) = layers[0]
    out = [_kron4(w) for w in jnp.split(w0, splits, axis=0)] + [_tile4(b0)]
    for w, b in layers[1:]:
        out += [_kron4(w), _tile4(b)]
    if with_ln:
        g, be = p["ln"]
        out += [_tile4(g), _tile4(be), _mavg()]
    return out


def _enc_node(x4, ws):
    def body(x_ref, d0, b0, d1, b1, d2, b2, g, be, m, o_ref):
        y = jnp.maximum(
            jnp.dot(x_ref[...], d0[...], preferred_element_type=jnp.float32)
            + b0[...], 0.0)
        y = jnp.maximum(
            jnp.dot(y, d1[...], preferred_element_type=jnp.float32) + b1[...], 0.0)
        y = jnp.dot(y, d2[...], preferred_element_type=jnp.float32) + b2[...]
        o_ref[...] = _ln_packed(y, g[...], be[...], m[...])

    return pl.pallas_call(
        body, out_shape=jax.ShapeDtypeStruct((_NP, 128), jnp.float32))(x4, *ws)


def _enc_edge1(ea128, d0x, b0x):
    def body(x_ref, d_ref, b_ref, o_ref):
        o_ref[...] = jnp.maximum(
            jnp.dot(x_ref[...], d_ref[...], preferred_element_type=jnp.float32)
            + b_ref[...], 0.0)

    return pl.pallas_call(
        body,
        grid=(10,),
        in_specs=[
            pl.BlockSpec((1000, 128), lambda i: (i, 0)),
            pl.BlockSpec(d0x.shape, lambda i: (0, 0)),
            pl.BlockSpec(b0x.shape, lambda i: (0, 0)),
        ],
        out_specs=pl.BlockSpec((1000, 1024), lambda i: (i, 0)),
        out_shape=jax.ShapeDtypeStruct((10000, 1024), jnp.float32),
    )(ea128, d0x, b0x)


def _enc_edge2(x1p, ws):
    def body(x_ref, d1, b1, d2, b2, g, be, m, o_ref):
        i = pl.program_id(0)
        y = jnp.maximum(
            jnp.dot(x_ref[...], d1[...], preferred_element_type=jnp.float32)
            + b1[...], 0.0)
        y = jnp.dot(y, d2[...], preferred_element_type=jnp.float32) + b2[...]
        y = _ln_packed(y, g[...], be[...], m[...])
        rows = i * 640 + lax.broadcasted_iota(jnp.int32, (640, 128), 0)
        o_ref[...] = jnp.where(rows < _VALID_P, y, 0.0)

    wspecs = [pl.BlockSpec(w.shape, lambda i: (0, 0)) for w in ws]
    return pl.pallas_call(
        body,
        grid=(_EP // 640,),
        in_specs=[pl.BlockSpec((640, 128), lambda i: (jnp.minimum(i, 124), 0))]
        + wspecs,
        out_specs=pl.BlockSpec((640, 128), lambda i: (i, 0)),
        out_shape=jax.ShapeDtypeStruct((_EP, 128), jnp.float32),
    )(x1p, *ws)


def _edge_step(e_p, gath_p, ws):
    def body(e_ref, hs_ref, hd_ref, d1e, d1s, d1d, b1, d2, b2, d3, b3,
             g, be, m, o_ref):
        i = pl.program_id(0)
        y = jnp.dot(e_ref[...], d1e[...], preferred_element_type=jnp.float32)
        y = y + jnp.dot(hs_ref[...], d1s[...], preferred_element_type=jnp.float32)
        y = y + jnp.dot(hd_ref[...], d1d[...], preferred_element_type=jnp.float32)
        y = jnp.maximum(y + b1[...], 0.0)
        y = jnp.maximum(
            jnp.dot(y, d2[...], preferred_element_type=jnp.float32) + b2[...], 0.0)
        y = jnp.dot(y, d3[...], preferred_element_type=jnp.float32) + b3[...]
        y = _ln_packed(y, g[...], be[...], m[...])
        rows = i * _EBP + lax.broadcasted_iota(jnp.int32, (_EBP, 128), 0)
        o_ref[...] = jnp.where(rows < _VALID_P, e_ref[...] + y, 0.0)

    wspecs = [pl.BlockSpec(w.shape, lambda i: (0, 0)) for w in ws]
    return pl.pallas_call(
        body,
        grid=(_N_EBLK,),
        in_specs=[
            pl.BlockSpec((_EBP, 128), lambda i: (i, 0)),
            pl.BlockSpec((_EBP, 128), lambda i: (i, 0)),
            pl.BlockSpec((_EBP, 128), lambda i: (i + _N_EBLK, 0)),
        ]
        + wspecs,
        out_specs=pl.BlockSpec((_EBP, 128), lambda i: (i, 0)),
        out_shape=jax.ShapeDtypeStruct((_EP, 128), jnp.float32),
    )(e_p, gath_p, gath_p, *ws)


def _node_step(h_p, agg_p, ws):
    def body(h_ref, a_ref, d1h, d1a, b1, d2, b2, d3, b3, g, be, m, o_ref):
        a = a_ref[pl.ds(0, _NP), :] + a_ref[pl.ds(_NP, _NP), :]
        y = jnp.dot(h_ref[...], d1h[...], preferred_element_type=jnp.float32)
        y = y + jnp.dot(a, d1a[...], preferred_element_type=jnp.float32)
        y = jnp.maximum(y + b1[...], 0.0)
        y = jnp.maximum(
            jnp.dot(y, d2[...], preferred_element_type=jnp.float32) + b2[...], 0.0)
        y = jnp.dot(y, d3[...], preferred_element_type=jnp.float32) + b3[...]
        o_ref[...] = h_ref[...] + _ln_packed(y, g[...], be[...], m[...])

    return pl.pallas_call(
        body, out_shape=jax.ShapeDtypeStruct((_NP, 128), jnp.float32))(
            h_p, agg_p, *ws)


def _dec(h_p, ws):
    def body(h_ref, d0, b0, d1, b1, d2, b2, o_ref):
        y = jnp.maximum(
            jnp.dot(h_ref[...], d0[...], preferred_element_type=jnp.float32)
            + b0[...], 0.0)
        y = jnp.maximum(
            jnp.dot(y, d1[...], preferred_element_type=jnp.float32) + b1[...], 0.0)
        o_ref[...] = jnp.dot(y, d2[...], preferred_element_type=jnp.float32) + b2[...]

    return pl.pallas_call(
        body, out_shape=jax.ShapeDtypeStruct((_NP, 12), jnp.float32))(h_p, *ws)


def _sc_mesh():
    return plsc.VectorSubcoreMesh(core_axis_name="c", subcore_axis_name="s")


def _sc_gather(h, idx2d):
    @functools.partial(
        pl.kernel,
        mesh=_sc_mesh(),
        compiler_params=pltpu.CompilerParams(use_tc_tiling_on_sc=False),
        out_type=jax.ShapeDtypeStruct((_IDX_PAD, _L), jnp.float32),
        scratch_types=[
            pltpu.VMEM((2, 8, 128), jnp.int32),
            pltpu.VMEM((2048, _L), jnp.float32),
            pltpu.SemaphoreType.DMA,
            pltpu.SemaphoreType.DMA,
            pltpu.SemaphoreType.DMA,
            pltpu.SemaphoreType.DMA,
        ],
    )
    def gk(h_hbm, idx_hbm, out_hbm, idx_v, rows_v, sg0, sg1, sw0, sw1):
        wid = lax.axis_index("s") * _NC + lax.axis_index("c")
        base_i = wid * 160
        base_o = wid * 20480
        sg = (sg0, sg1)
        sw = (sw0, sw1)

        def fire(t, b):
            pltpu.sync_copy(idx_hbm.at[pl.ds(base_i + t * 8, 8)], idx_v.at[b])
            for j in range(8):
                pltpu.async_copy(
                    h_hbm.at[idx_v.at[b, j]],
                    rows_v.at[pl.ds(b * 1024 + j * 128, 128)], sg[b])

        def drain_g(b):
            for j in range(8):
                pltpu.make_async_copy(
                    h_hbm.at[idx_v.at[b, j]],
                    rows_v.at[pl.ds(b * 1024 + j * 128, 128)], sg[b]).wait()

        def wb(t, b):
            pltpu.async_copy(
                rows_v.at[pl.ds(b * 1024, 1024)],
                out_hbm.at[pl.ds(base_o + t * 1024, 1024)], sw[b])

        def drain_w(t, b):
            pltpu.make_async_copy(
                rows_v.at[pl.ds(b * 1024, 1024)],
                out_hbm.at[pl.ds(base_o + t * 1024, 1024)], sw[b]).wait()

        fire(0, 0)

        def body(g, c):
            @pl.when(g > 0)
            def _():
                drain_w(2 * g - 1, 1)

            fire(2 * g + 1, 1)
            drain_g(0)
            wb(2 * g, 0)
            drain_w(2 * g, 0)
            fire(2 * g + 2, 0)
            drain_g(1)
            wb(2 * g + 1, 1)
            return c

        lax.fori_loop(0, 9, body, 0)
        drain_w(17, 1)
        fire(19, 1)
        drain_g(0)
        wb(18, 0)
        drain_g(1)
        wb(19, 1)
        drain_w(18, 0)
        drain_w(19, 1)

    return gk(h, idx2d)


def _sc_scatter(e, didx2d, zblk):
    @functools.partial(
        pl.kernel,
        mesh=_sc_mesh(),
        compiler_params=pltpu.CompilerParams(use_tc_tiling_on_sc=False),
        out_type=jax.ShapeDtypeStruct((2 * _N, _L), jnp.float32),
        scratch_types=[
            pltpu.VMEM_SHARED((10240, _L), jnp.float32),
            pltpu.VMEM((8, 128), jnp.int32),
            pltpu.VMEM((1024, _L), jnp.float32),
            pltpu.SemaphoreType.DMA,
        ],
    )
    def sk(e_hbm, idx_hbm, z_hbm, out_hbm, shared, idx_v, rows_v, sem):
        core = lax.axis_index("c")
        sub = lax.axis_index("s")
        wid = sub * _NC + core
        pltpu.sync_copy(z_hbm, shared.at[pl.ds(sub * 640, 640)])
        plsc.subcore_barrier()

        def step(t, c):
            pltpu.sync_copy(e_hbm.at[pl.ds(wid * 10240 + t * 1024, 1024)], rows_v)
            pltpu.sync_copy(idx_hbm.at[pl.ds(wid * 80 + t * 8, 8)], idx_v)
            for j in range(8):
                pltpu.sync_copy(
                    rows_v.at[pl.ds(j * 128, 128)],
                    shared.at[idx_v.at[j]],
                    add=True,
                )
            return c

        lax.fori_loop(0, 10, step, 0)
        plsc.subcore_barrier()
        pltpu.sync_copy(
            shared.at[pl.ds(sub * 625, 625)],
            out_hbm.at[pl.ds(core * _N + sub * 625, 625)],
        )

    return sk(e, didx2d, zblk)


def kernel(x, edge_index, edge_attr, params):
    zpad = jnp.zeros((_E_PAD - _E,), jnp.int32)
    gidx = jnp.concatenate(
        [edge_index[0], zpad, edge_index[1], zpad]).reshape(_IDX_PAD // 128, 128)
    didx = jnp.concatenate([edge_index[1], zpad]).reshape(_E_PAD // 128, 128)
    zblk = jnp.zeros((640, _L), jnp.float32)

    h_p = _enc_node(x.reshape(_NP, 512), _prep(params["enc_node"], 1))
    pe = params["enc_edge"]
    (w0, b0), (w1, b1), (w2, b2) = pe["layers"]
    g, be = pe["ln"]
    e1 = _enc_edge1(
        edge_attr.reshape(10000, 128),
        jnp.kron(jnp.eye(32, dtype=jnp.float32), w0),
        jnp.tile(b0, 32).reshape(1, -1))
    e_p = _enc_edge2(
        e1.reshape(_VALID_P, 128),
        [_kron4(w1), _tile4(b1), _kron4(w2), _tile4(b2),
         _tile4(g), _tile4(be), _mavg()])
    for blk in params["processor"]:
        ews = _prep(blk["edge"], 3)
        nws = _prep(blk["node"], 2)
        gath = _sc_gather(h_p.reshape(_N, _L), gidx)
        e_p = _edge_step(e_p, gath.reshape(2 * _EP, 128), ews)
        agg = _sc_scatter(e_p.reshape(_E_PAD, _L), didx, zblk)
        h_p = _node_step(h_p, agg.reshape(2 * _NP, 128), nws)
    return _dec(h_p, _prep(params["dec"], 1, with_ln=False)).reshape(_N, 3)
